# baseline (speedup 1.0000x reference)
import functools

import jax
import jax.numpy as jnp
from jax import lax
from jax.experimental import pallas as pl
from jax.experimental.pallas import tpu as pltpu
from jax.experimental.pallas import tpu_sc as plsc

VOCAB = 100000
HIDDEN = 1024
NTOK = 16384

NC = 2
NS = 16
NW = NC * NS
BPW = NTOK // NW
CHUNK = 16
NCHUNK = BPW // CHUNK
NBUF = 3

_mesh = plsc.VectorSubcoreMesh(core_axis_name="c", subcore_axis_name="s")


@functools.partial(
    pl.kernel,
    out_type=jax.ShapeDtypeStruct((NTOK, HIDDEN), jnp.float32),
    mesh=_mesh,
    scratch_types=[
        pltpu.VMEM((NCHUNK, CHUNK), jnp.int32),
        pltpu.VMEM((NBUF, CHUNK, HIDDEN), jnp.float32),
        pltpu.VMEM_SHARED((NS, NBUF, CHUNK, HIDDEN), jnp.float32),
        pltpu.SemaphoreType.DMA,
        pltpu.SemaphoreType.DMA,
        pltpu.SemaphoreType.DMA,
        pltpu.SemaphoreType.DMA,
        pltpu.SemaphoreType.DMA,
        pltpu.SemaphoreType.DMA,
        pltpu.SemaphoreType.DMA,
        pltpu.SemaphoreType.DMA,
        pltpu.SemaphoreType.DMA,
    ],
)
def _emb_lookup(x_hbm, table_hbm, out_hbm, idx_v, bufs, spmem,
                gsem0, gsem1, gsem2, csem0, csem1, csem2,
                ssem0, ssem1, ssem2):
    wid = lax.axis_index("s") * NC + lax.axis_index("c")
    sid = lax.axis_index("s")
    base = wid * BPW

    pltpu.sync_copy(x_hbm.at[wid], idx_v)

    gsems = (gsem0, gsem1, gsem2)
    csems = (csem0, csem1, csem2)
    ssems = (ssem0, ssem1, ssem2)

    def gather(g):
        return pltpu.async_copy(
            table_hbm.at[idx_v.at[g]], bufs.at[g % NBUF], gsems[g % NBUF])

    def to_spmem(g):
        return pltpu.async_copy(
            bufs.at[g % NBUF], spmem.at[sid, g % NBUF], csems[g % NBUF])

    def to_hbm(g):
        return pltpu.async_copy(
            spmem.at[sid, g % NBUF],
            out_hbm.at[pl.ds(base + g * CHUNK, CHUNK)],
            ssems[g % NBUF])

    copies_g = [None] * NCHUNK
    copies_c = [None] * NCHUNK
    copies_s = [None] * NCHUNK
    for g in range(NBUF):
        copies_g[g] = gather(g)
    for g in range(NCHUNK):
        copies_g[g].wait()
        if g >= NBUF:
            copies_s[g - NBUF].wait()
        copies_c[g] = to_spmem(g)
        copies_c[g].wait()
        copies_s[g] = to_hbm(g)
        if g + NBUF < NCHUNK:
            copies_g[g + NBUF] = gather(g + NBUF)
    for g in range(NCHUNK - NBUF, NCHUNK):
        copies_s[g].wait()


def kernel(x, table):
    ids = x.reshape(NW, NCHUNK, CHUNK).astype(jnp.int32)
    out = _emb_lookup(ids, table)
    return out.reshape(x.shape[0], x.shape[1], HIDDEN)
